# baseline (speedup 1.0000x reference)
import jax
import jax.numpy as jnp
from jax.experimental import pallas as pl

M = 8


def _noop_body(x_ref, o_ref):
    o_ref[...] = x_ref[...]


def kernel(x_input):
    L = x_input.shape[1]
    x_DFT = jnp.fft.fft(x_input, axis=1)
    a = jnp.abs(x_DFT)
    b, _, d = a.shape
    flat = a.reshape(b, 8, (L * d) // 8)
    flat = pl.pallas_call(
        _noop_body,
        grid=(b,),
        in_specs=[pl.BlockSpec((1, 8, (L * d) // 8), lambda i: (i, 0, 0))],
        out_specs=pl.BlockSpec((1, 8, (L * d) // 8), lambda i: (i, 0, 0)),
        out_shape=jax.ShapeDtypeStruct(flat.shape, flat.dtype),
    )(flat)
    a = flat.reshape(b, L, d)
    a_m = jnp.moveaxis(a, 1, 2)
    vals, idx = jax.lax.top_k(a_m, M)
    f = jnp.moveaxis(idx, 2, 1) + 1
    p = jnp.ceil(L / f).astype(jnp.int64)
    return p


# speedup vs baseline: 6.7913x; 6.7913x over previous
import jax
import jax.numpy as jnp
from jax.experimental import pallas as pl

M = 8
L = 8192
COLS_PER_BLOCK = 128


def _topk_body(a_ref, p_ref):
    a = a_ref[...]
    rows = jax.lax.broadcasted_iota(jnp.int32, a.shape, 0)
    big = jnp.int32(L)
    for m in range(M):
        vmax = jnp.max(a, axis=0, keepdims=True)
        idx = jnp.min(jnp.where(a == vmax, rows, big), axis=0)
        f = (idx + 1).astype(jnp.float32)
        p_ref[m, :] = jnp.ceil(jnp.float32(L) / f).astype(jnp.int32)
        a = jnp.where(rows == idx[None, :], jnp.float32(-1.0), a)


def kernel(x_input):
    b, length, d = x_input.shape
    x_DFT = jnp.fft.fft(x_input, axis=1)
    a = jnp.abs(x_DFT)
    a_t = jnp.transpose(a, (1, 0, 2)).reshape(length, b * d)
    n_cols = b * d
    grid = n_cols // COLS_PER_BLOCK
    p = pl.pallas_call(
        _topk_body,
        grid=(grid,),
        in_specs=[pl.BlockSpec((length, COLS_PER_BLOCK), lambda j: (0, j))],
        out_specs=pl.BlockSpec((M, COLS_PER_BLOCK), lambda j: (0, j)),
        out_shape=jax.ShapeDtypeStruct((M, n_cols), jnp.int32),
    )(a_t)
    p = jnp.transpose(p.reshape(M, b, d), (1, 0, 2))
    return p.astype(jnp.int64)
